# Initial kernel scaffold; baseline (speedup 1.0000x reference)
import jax
import jax.numpy as jnp
from jax.experimental import pallas as pl


def kernel(x, edge_index, edge_weight, batch, W_rel, b_rel, W_root, W_fc, b_fc):
    raise NotImplementedError("write your pallas kernel here")


# speedup vs baseline: 28.6566x; 28.6566x over previous
import functools

import jax
import jax.numpy as jnp
from jax import lax
from jax.experimental import pallas as pl
from jax.experimental.pallas import tpu as pltpu
from jax.experimental.pallas import tpu_sc as plsc

N_NODES_C = 10000
N_PAD = 10240
N_EDGES_C = 320000
N_GRAPHS_C = 64
NT = 16
E_T = N_EDGES_C // NT
NODES_T = N_PAD // NT
L = 16


def _proj_body(x_ref, wrel_ref, wroot_ref, brel_ref, prel_ref, q_ref):
    x = x_ref[...]
    dn = (((1,), (1,)), ((), ()))
    prel_ref[...] = lax.dot_general(wrel_ref[...], x, dn,
                                    preferred_element_type=jnp.float32)
    q_ref[...] = lax.dot_general(wroot_ref[...], x, dn,
                                 preferred_element_type=jnp.float32) + brel_ref[...]


def _sc_body(prel_hbm, q_hbm, src_hbm, dst_hbm, ew_hbm, batch_hbm,
             wfc_hbm, bfc_hbm, out_hbm,
             src_v, dst_v, ew_v, pv_v, ones_v,
             stage_v, a_v, d_v, q_v, h_v, batch_v,
             pool_v, cnt_v, y_v, wfc_v, bfc_v,
             p_s, q_s, agg_s, deg_s, pool_s, cnt_s):
    wid = lax.axis_index("s")
    nbase = wid * NODES_T
    ebase = wid * E_T

    def _fill_zero(i, _):
        stage_v[pl.ds(i * L, L)] = jnp.zeros((L,), jnp.float32)
        return 0
    lax.fori_loop(0, NODES_T // L, _fill_zero, 0)

    def _fill_one(i, _):
        ones_v[pl.ds(i * L, L)] = jnp.ones((L,), jnp.float32)
        return 0
    lax.fori_loop(0, E_T // L, _fill_one, 0)

    pltpu.sync_copy(stage_v, agg_s.at[pl.ds(nbase, NODES_T)])
    pltpu.sync_copy(stage_v, deg_s.at[pl.ds(nbase, NODES_T)])

    @pl.when(wid == 0)
    def _zero_pool():
        pltpu.sync_copy(stage_v.at[pl.ds(0, 2 * N_GRAPHS_C)], pool_s)
        pltpu.sync_copy(stage_v.at[pl.ds(0, 2 * N_GRAPHS_C)], cnt_s)

    pltpu.sync_copy(prel_hbm.at[pl.ds(nbase, NODES_T)], a_v)
    pltpu.sync_copy(a_v, p_s.at[pl.ds(nbase, NODES_T)])
    pltpu.sync_copy(q_hbm.at[pl.ds(nbase, NODES_T)], d_v)
    pltpu.sync_copy(d_v, q_s.at[pl.ds(nbase, NODES_T)])

    pltpu.sync_copy(src_hbm.at[pl.ds(ebase, E_T)], src_v)
    pltpu.sync_copy(dst_hbm.at[pl.ds(ebase, E_T)], dst_v)
    pltpu.sync_copy(ew_hbm.at[pl.ds(ebase, E_T)], ew_v)

    plsc.subcore_barrier()

    pltpu.sync_copy(p_s.at[src_v], pv_v)

    def _msg(i, _):
        sl = pl.ds(i * L, L)
        pv_v[sl] = pv_v[sl] * ew_v[sl]
        return 0
    lax.fori_loop(0, E_T // L, _msg, 0)

    pltpu.sync_copy(pv_v, agg_s.at[dst_v], add=True)
    pltpu.sync_copy(ones_v, deg_s.at[dst_v], add=True)

    plsc.subcore_barrier()

    pltpu.sync_copy(agg_s.at[pl.ds(nbase, NODES_T)], a_v)
    pltpu.sync_copy(deg_s.at[pl.ds(nbase, NODES_T)], d_v)
    pltpu.sync_copy(q_s.at[pl.ds(nbase, NODES_T)], q_v)
    pltpu.sync_copy(batch_hbm.at[pl.ds(nbase, NODES_T)], batch_v)

    def _node(i, _):
        sl = pl.ds(i * L, L)
        z = a_v[sl] / jnp.maximum(d_v[sl], 1.0) + q_v[sl]
        h_v[sl] = jnp.maximum(z, 0.0)
        return 0
    lax.fori_loop(0, NODES_T // L, _node, 0)

    pltpu.sync_copy(h_v, pool_s.at[batch_v], add=True)
    pltpu.sync_copy(ones_v.at[pl.ds(0, NODES_T)], cnt_s.at[batch_v], add=True)

    plsc.subcore_barrier()

    @pl.when(wid == 0)
    def _final():
        pltpu.sync_copy(pool_s.at[pl.ds(0, N_GRAPHS_C)], pool_v)
        pltpu.sync_copy(cnt_s.at[pl.ds(0, N_GRAPHS_C)], cnt_v)
        pltpu.sync_copy(wfc_hbm, wfc_v)
        pltpu.sync_copy(bfc_hbm, bfc_v)
        wfc = wfc_v[...]
        bfc = bfc_v[...]
        for j in range(N_GRAPHS_C // L):
            sl = pl.ds(j * L, L)
            z = pool_v[sl] / jnp.maximum(cnt_v[sl], 1.0) * wfc + bfc
            y_v[sl] = 1.0 / (1.0 + jnp.exp(-z))
        pltpu.sync_copy(y_v, out_hbm)


_sc_call = pl.kernel(
    _sc_body,
    out_type=jax.ShapeDtypeStruct((N_GRAPHS_C,), jnp.float32),
    mesh=plsc.VectorSubcoreMesh(core_axis_name="c", subcore_axis_name="s",
                                num_cores=1),
    scratch_types=[
        pltpu.VMEM((E_T,), jnp.int32),
        pltpu.VMEM((E_T,), jnp.int32),
        pltpu.VMEM((E_T,), jnp.float32),
        pltpu.VMEM((E_T,), jnp.float32),
        pltpu.VMEM((E_T,), jnp.float32),
        pltpu.VMEM((NODES_T,), jnp.float32),
        pltpu.VMEM((NODES_T,), jnp.float32),
        pltpu.VMEM((NODES_T,), jnp.float32),
        pltpu.VMEM((NODES_T,), jnp.float32),
        pltpu.VMEM((NODES_T,), jnp.float32),
        pltpu.VMEM((NODES_T,), jnp.int32),
        pltpu.VMEM((N_GRAPHS_C,), jnp.float32),
        pltpu.VMEM((N_GRAPHS_C,), jnp.float32),
        pltpu.VMEM((N_GRAPHS_C,), jnp.float32),
        pltpu.VMEM((L,), jnp.float32),
        pltpu.VMEM((L,), jnp.float32),
        pltpu.VMEM_SHARED((N_PAD,), jnp.float32),
        pltpu.VMEM_SHARED((N_PAD,), jnp.float32),
        pltpu.VMEM_SHARED((N_PAD,), jnp.float32),
        pltpu.VMEM_SHARED((N_PAD,), jnp.float32),
        pltpu.VMEM_SHARED((2 * N_GRAPHS_C,), jnp.float32),
        pltpu.VMEM_SHARED((2 * N_GRAPHS_C,), jnp.float32),
    ],
)


@jax.jit
def kernel(x, edge_index, edge_weight, batch, W_rel, b_rel, W_root, W_fc, b_fc):
    src = edge_index[0].astype(jnp.int32)
    dst = edge_index[1].astype(jnp.int32)
    batch_pad = jnp.concatenate(
        [batch.astype(jnp.int32),
         jnp.full((N_PAD - N_NODES_C,), N_GRAPHS_C, jnp.int32)])
    x_pad = jnp.pad(x, ((0, N_PAD - N_NODES_C), (0, 0)))

    prel2, q2 = pl.pallas_call(
        _proj_body,
        out_shape=[jax.ShapeDtypeStruct((1, N_PAD), jnp.float32),
                   jax.ShapeDtypeStruct((1, N_PAD), jnp.float32)],
    )(x_pad, W_rel, W_root, jnp.reshape(b_rel, (1, 1)))

    wfc_b = jnp.full((L,), W_fc[0, 0], jnp.float32)
    bfc_b = jnp.full((L,), b_fc[0], jnp.float32)

    y = _sc_call(prel2.reshape(N_PAD), q2.reshape(N_PAD),
                 src, dst, edge_weight, batch_pad, wfc_b, bfc_b)
    return y[:, None]


# speedup vs baseline: 38.1767x; 1.3322x over previous
import jax
import jax.numpy as jnp
from jax import lax
from jax.experimental import pallas as pl
from jax.experimental.pallas import tpu as pltpu
from jax.experimental.pallas import tpu_sc as plsc

N_NODES_C = 10000
N_PAD = 10240
N_EDGES_C = 320000
N_GRAPHS_C = 64
NT = 16
E_T = N_EDGES_C // NT
NODES_T = N_PAD // NT
L = 16


def _proj_body(x_ref, wrel_ref, wroot_ref, brel_ref, prel_ref, q_ref):
    x = x_ref[...]
    dn = (((1,), (1,)), ((), ()))
    pr = lax.dot_general(wrel_ref[...], x, dn,
                         preferred_element_type=jnp.float32)
    qq = lax.dot_general(wroot_ref[...], x, dn,
                         preferred_element_type=jnp.float32) + brel_ref[...]
    prel_ref[:, pl.ds(0, N_NODES_C)] = pr
    q_ref[:, pl.ds(0, N_NODES_C)] = qq
    zt = jnp.zeros((1, N_PAD - N_NODES_C), jnp.float32)
    prel_ref[:, pl.ds(N_NODES_C, N_PAD - N_NODES_C)] = zt
    q_ref[:, pl.ds(N_NODES_C, N_PAD - N_NODES_C)] = zt


def _sc_body(prel_hbm, q_hbm, ei_hbm, ew_hbm, batch_hbm, par_hbm,
             out_hbm,
             src_v, dst_v, ew_v, pv_v, ones_v,
             stage_v, a_v, d_v, q_v, h_v, batch_v,
             pool_v, cnt_v, y_v, par_v,
             s_src, s_dst, s_ew, s_p, s_q, s_b,
             p_s, q_s, agg_s, deg_s, pool_s, cnt_s):
    wid = lax.axis_index("s")
    nbase = wid * NODES_T
    ebase = wid * E_T

    cp_src = pltpu.async_copy(ei_hbm.at[pl.ds(ebase, E_T)], src_v, s_src)
    cp_dst = pltpu.async_copy(ei_hbm.at[pl.ds(N_EDGES_C + ebase, E_T)],
                              dst_v, s_dst)
    cp_ew = pltpu.async_copy(ew_hbm.at[pl.ds(ebase, E_T)], ew_v, s_ew)
    cp_p = pltpu.async_copy(prel_hbm.at[pl.ds(nbase, NODES_T)], a_v, s_p)
    cp_q = pltpu.async_copy(q_hbm.at[pl.ds(nbase, NODES_T)], d_v, s_q)
    cp_b = pltpu.async_copy(batch_hbm.at[pl.ds(nbase, NODES_T)], batch_v, s_b)

    def _fill_zero(i, _):
        stage_v[pl.ds(i * L, L)] = jnp.zeros((L,), jnp.float32)
        return 0
    lax.fori_loop(0, NODES_T // L, _fill_zero, 0)

    def _fill_one(i, _):
        ones_v[pl.ds(i * L, L)] = jnp.ones((L,), jnp.float32)
        return 0
    lax.fori_loop(0, E_T // L, _fill_one, 0)

    pltpu.sync_copy(stage_v, agg_s.at[pl.ds(nbase, NODES_T)])
    pltpu.sync_copy(stage_v, deg_s.at[pl.ds(nbase, NODES_T)])

    @pl.when(wid == 0)
    def _zero_pool():
        pltpu.sync_copy(stage_v.at[pl.ds(0, 2 * N_GRAPHS_C)], pool_s)
        pltpu.sync_copy(stage_v.at[pl.ds(0, 2 * N_GRAPHS_C)], cnt_s)

    cp_p.wait()
    pltpu.sync_copy(a_v, p_s.at[pl.ds(nbase, NODES_T)])
    cp_q.wait()
    pltpu.sync_copy(d_v, q_s.at[pl.ds(nbase, NODES_T)])
    cp_src.wait()
    cp_dst.wait()
    cp_ew.wait()
    cp_b.wait()

    plsc.subcore_barrier()

    pltpu.sync_copy(p_s.at[src_v], pv_v)

    def _msg(i, _):
        sl = pl.ds(i * L, L)
        pv_v[sl] = pv_v[sl] * ew_v[sl]
        return 0
    lax.fori_loop(0, E_T // L, _msg, 0)

    pltpu.sync_copy(pv_v, agg_s.at[dst_v], add=True)
    pltpu.sync_copy(ones_v, deg_s.at[dst_v], add=True)

    plsc.subcore_barrier()

    pltpu.sync_copy(agg_s.at[pl.ds(nbase, NODES_T)], a_v)
    pltpu.sync_copy(deg_s.at[pl.ds(nbase, NODES_T)], d_v)
    pltpu.sync_copy(q_s.at[pl.ds(nbase, NODES_T)], q_v)

    def _node(i, _):
        sl = pl.ds(i * L, L)
        z = a_v[sl] / jnp.maximum(d_v[sl], 1.0) + q_v[sl]
        h_v[sl] = jnp.maximum(z, 0.0)
        return 0
    lax.fori_loop(0, NODES_T // L, _node, 0)

    pltpu.sync_copy(h_v, pool_s.at[batch_v], add=True)
    pltpu.sync_copy(ones_v.at[pl.ds(0, NODES_T)], cnt_s.at[batch_v], add=True)

    plsc.subcore_barrier()

    @pl.when(wid == 0)
    def _final():
        pltpu.sync_copy(pool_s.at[pl.ds(0, N_GRAPHS_C)], pool_v)
        pltpu.sync_copy(cnt_s.at[pl.ds(0, N_GRAPHS_C)], cnt_v)
        pltpu.sync_copy(par_hbm, par_v)
        wfc = par_v[pl.ds(0, L)]
        bfc = par_v[pl.ds(L, L)]
        for j in range(N_GRAPHS_C // L):
            sl = pl.ds(j * L, L)
            z = pool_v[sl] / jnp.maximum(cnt_v[sl], 1.0) * wfc + bfc
            y_v[sl] = 1.0 / (1.0 + jnp.exp(-z))
        pltpu.sync_copy(y_v, out_hbm)


_sc_call = pl.kernel(
    _sc_body,
    out_type=jax.ShapeDtypeStruct((N_GRAPHS_C,), jnp.float32),
    mesh=plsc.VectorSubcoreMesh(core_axis_name="c", subcore_axis_name="s",
                                num_cores=1),
    scratch_types=[
        pltpu.VMEM((E_T,), jnp.int32),
        pltpu.VMEM((E_T,), jnp.int32),
        pltpu.VMEM((E_T,), jnp.float32),
        pltpu.VMEM((E_T,), jnp.float32),
        pltpu.VMEM((E_T,), jnp.float32),
        pltpu.VMEM((NODES_T,), jnp.float32),
        pltpu.VMEM((NODES_T,), jnp.float32),
        pltpu.VMEM((NODES_T,), jnp.float32),
        pltpu.VMEM((NODES_T,), jnp.float32),
        pltpu.VMEM((NODES_T,), jnp.float32),
        pltpu.VMEM((NODES_T,), jnp.int32),
        pltpu.VMEM((N_GRAPHS_C,), jnp.float32),
        pltpu.VMEM((N_GRAPHS_C,), jnp.float32),
        pltpu.VMEM((N_GRAPHS_C,), jnp.float32),
        pltpu.VMEM((2 * L,), jnp.float32),
        pltpu.SemaphoreType.DMA,
        pltpu.SemaphoreType.DMA,
        pltpu.SemaphoreType.DMA,
        pltpu.SemaphoreType.DMA,
        pltpu.SemaphoreType.DMA,
        pltpu.SemaphoreType.DMA,
        pltpu.VMEM_SHARED((N_PAD,), jnp.float32),
        pltpu.VMEM_SHARED((N_PAD,), jnp.float32),
        pltpu.VMEM_SHARED((N_PAD,), jnp.float32),
        pltpu.VMEM_SHARED((N_PAD,), jnp.float32),
        pltpu.VMEM_SHARED((2 * N_GRAPHS_C,), jnp.float32),
        pltpu.VMEM_SHARED((2 * N_GRAPHS_C,), jnp.float32),
    ],
)


@jax.jit
def kernel(x, edge_index, edge_weight, batch, W_rel, b_rel, W_root, W_fc, b_fc):
    ei_flat = edge_index.astype(jnp.int32).reshape(2 * N_EDGES_C)
    batch_pad = jnp.concatenate(
        [batch.astype(jnp.int32),
         jnp.full((N_PAD - N_NODES_C,), N_GRAPHS_C, jnp.int32)])

    prel2, q2 = pl.pallas_call(
        _proj_body,
        out_shape=[jax.ShapeDtypeStruct((1, N_PAD), jnp.float32),
                   jax.ShapeDtypeStruct((1, N_PAD), jnp.float32)],
    )(x, W_rel, W_root, jnp.reshape(b_rel, (1, 1)))

    params = jnp.concatenate([jnp.broadcast_to(W_fc[0], (L,)),
                              jnp.broadcast_to(b_fc, (L,))])
    y = _sc_call(prel2.reshape(N_PAD), q2.reshape(N_PAD),
                 ei_flat, edge_weight, batch_pad, params)
    return y[:, None]


# speedup vs baseline: 39.3013x; 1.0295x over previous
import jax
import jax.numpy as jnp
from jax import lax
from jax.experimental import pallas as pl
from jax.experimental.pallas import tpu as pltpu
from jax.experimental.pallas import tpu_sc as plsc

N_NODES_C = 10000
N_PAD = 10240
N_EDGES_C = 320000
N_GRAPHS_C = 64
NT = 16
E_T = N_EDGES_C // NT
NODES_T = N_PAD // NT
L = 16


def _proj_body(x_ref, wrel_ref, wroot_ref, brel_ref, batch_ref, wfc_ref,
               bfc_ref, prel_ref, q_ref, bp_ref, par_ref):
    x = x_ref[...]
    dn = (((1,), (1,)), ((), ()))
    pr = lax.dot_general(wrel_ref[...], x, dn,
                         preferred_element_type=jnp.float32)
    qq = lax.dot_general(wroot_ref[...], x, dn,
                         preferred_element_type=jnp.float32) + brel_ref[...]
    prel_ref[:, pl.ds(0, N_NODES_C)] = pr
    q_ref[:, pl.ds(0, N_NODES_C)] = qq
    zt = jnp.zeros((1, N_PAD - N_NODES_C), jnp.float32)
    prel_ref[:, pl.ds(N_NODES_C, N_PAD - N_NODES_C)] = zt
    q_ref[:, pl.ds(N_NODES_C, N_PAD - N_NODES_C)] = zt
    bp_ref[:, pl.ds(0, N_NODES_C)] = jnp.reshape(batch_ref[...],
                                                 (1, N_NODES_C))
    bp_ref[:, pl.ds(N_NODES_C, N_PAD - N_NODES_C)] = jnp.full(
        (1, N_PAD - N_NODES_C), N_GRAPHS_C, jnp.int32)
    par_ref[...] = jnp.concatenate(
        [jnp.broadcast_to(wfc_ref[...], (1, L)),
         jnp.broadcast_to(jnp.reshape(bfc_ref[...], (1, 1)), (1, L)),
         jnp.zeros((1, 128 - 2 * L), jnp.float32)], axis=1)


def _sc_body(prel_hbm, q_hbm, ei_hbm, ew_hbm, bp_hbm, par_hbm,
             out_hbm,
             src_v, dst_v, ew_v, pv_v, ones_v,
             stage_v, a_v, d_v, q_v, h_v, batch_v,
             pool_v, cnt_v, y_v, par_v,
             s_src, s_dst, s_ew, s_p, s_q, s_b,
             p_s, q_s, agg_s, deg_s, pool_s, cnt_s):
    wid = lax.axis_index("s")
    nbase = wid * NODES_T
    ebase = wid * E_T

    cp_src = pltpu.async_copy(ei_hbm.at[pl.ds(ebase, E_T)], src_v, s_src)
    cp_dst = pltpu.async_copy(ei_hbm.at[pl.ds(N_EDGES_C + ebase, E_T)],
                              dst_v, s_dst)
    cp_ew = pltpu.async_copy(ew_hbm.at[pl.ds(ebase, E_T)], ew_v, s_ew)
    cp_p = pltpu.async_copy(prel_hbm.at[pl.ds(nbase, NODES_T)], a_v, s_p)
    cp_q = pltpu.async_copy(q_hbm.at[pl.ds(nbase, NODES_T)], d_v, s_q)
    cp_b = pltpu.async_copy(bp_hbm.at[pl.ds(nbase, NODES_T)], batch_v, s_b)

    def _fill_zero(i, _):
        stage_v[pl.ds(i * L, L)] = jnp.zeros((L,), jnp.float32)
        return 0
    lax.fori_loop(0, NODES_T // L, _fill_zero, 0)

    def _fill_one(i, _):
        ones_v[pl.ds(i * L, L)] = jnp.ones((L,), jnp.float32)
        return 0
    lax.fori_loop(0, E_T // L, _fill_one, 0)

    pltpu.sync_copy(stage_v, agg_s.at[pl.ds(nbase, NODES_T)])
    pltpu.sync_copy(stage_v, deg_s.at[pl.ds(nbase, NODES_T)])

    @pl.when(wid == 0)
    def _zero_pool():
        pltpu.sync_copy(stage_v.at[pl.ds(0, 2 * N_GRAPHS_C)], pool_s)
        pltpu.sync_copy(stage_v.at[pl.ds(0, 2 * N_GRAPHS_C)], cnt_s)

    cp_p.wait()
    pltpu.sync_copy(a_v, p_s.at[pl.ds(nbase, NODES_T)])
    cp_q.wait()
    pltpu.sync_copy(d_v, q_s.at[pl.ds(nbase, NODES_T)])
    cp_src.wait()
    cp_dst.wait()
    cp_ew.wait()
    cp_b.wait()

    plsc.subcore_barrier()

    pltpu.sync_copy(p_s.at[src_v], pv_v)

    def _msg(i, _):
        sl = pl.ds(i * L, L)
        pv_v[sl] = pv_v[sl] * ew_v[sl]
        return 0
    lax.fori_loop(0, E_T // L, _msg, 0)

    cs_a = pltpu.async_copy(pv_v, agg_s.at[dst_v], s_p, add=True)
    cs_d = pltpu.async_copy(ones_v, deg_s.at[dst_v], s_q, add=True)
    cs_a.wait()
    cs_d.wait()

    plsc.subcore_barrier()

    cn_a = pltpu.async_copy(agg_s.at[pl.ds(nbase, NODES_T)], a_v, s_p)
    cn_d = pltpu.async_copy(deg_s.at[pl.ds(nbase, NODES_T)], d_v, s_q)
    cn_q = pltpu.async_copy(q_s.at[pl.ds(nbase, NODES_T)], q_v, s_b)
    cn_a.wait()
    cn_d.wait()
    cn_q.wait()

    def _node(i, _):
        sl = pl.ds(i * L, L)
        z = a_v[sl] / jnp.maximum(d_v[sl], 1.0) + q_v[sl]
        h_v[sl] = jnp.maximum(z, 0.0)
        return 0
    lax.fori_loop(0, NODES_T // L, _node, 0)

    cp_pool = pltpu.async_copy(h_v, pool_s.at[batch_v], s_p, add=True)
    cp_cnt = pltpu.async_copy(ones_v.at[pl.ds(0, NODES_T)],
                              cnt_s.at[batch_v], s_q, add=True)
    cp_pool.wait()
    cp_cnt.wait()

    plsc.subcore_barrier()

    @pl.when(wid == 0)
    def _final():
        pltpu.sync_copy(pool_s.at[pl.ds(0, N_GRAPHS_C)], pool_v)
        pltpu.sync_copy(cnt_s.at[pl.ds(0, N_GRAPHS_C)], cnt_v)
        pltpu.sync_copy(par_hbm.at[pl.ds(0, 2 * L)], par_v)
        wfc = par_v[pl.ds(0, L)]
        bfc = par_v[pl.ds(L, L)]
        for j in range(N_GRAPHS_C // L):
            sl = pl.ds(j * L, L)
            z = pool_v[sl] / jnp.maximum(cnt_v[sl], 1.0) * wfc + bfc
            y_v[sl] = 1.0 / (1.0 + jnp.exp(-z))
        pltpu.sync_copy(y_v, out_hbm)


_sc_call = pl.kernel(
    _sc_body,
    out_type=jax.ShapeDtypeStruct((N_GRAPHS_C,), jnp.float32),
    mesh=plsc.VectorSubcoreMesh(core_axis_name="c", subcore_axis_name="s",
                                num_cores=1),
    scratch_types=[
        pltpu.VMEM((E_T,), jnp.int32),
        pltpu.VMEM((E_T,), jnp.int32),
        pltpu.VMEM((E_T,), jnp.float32),
        pltpu.VMEM((E_T,), jnp.float32),
        pltpu.VMEM((E_T,), jnp.float32),
        pltpu.VMEM((NODES_T,), jnp.float32),
        pltpu.VMEM((NODES_T,), jnp.float32),
        pltpu.VMEM((NODES_T,), jnp.float32),
        pltpu.VMEM((NODES_T,), jnp.float32),
        pltpu.VMEM((NODES_T,), jnp.float32),
        pltpu.VMEM((NODES_T,), jnp.int32),
        pltpu.VMEM((N_GRAPHS_C,), jnp.float32),
        pltpu.VMEM((N_GRAPHS_C,), jnp.float32),
        pltpu.VMEM((N_GRAPHS_C,), jnp.float32),
        pltpu.VMEM((2 * L,), jnp.float32),
        pltpu.SemaphoreType.DMA,
        pltpu.SemaphoreType.DMA,
        pltpu.SemaphoreType.DMA,
        pltpu.SemaphoreType.DMA,
        pltpu.SemaphoreType.DMA,
        pltpu.SemaphoreType.DMA,
        pltpu.VMEM_SHARED((N_PAD,), jnp.float32),
        pltpu.VMEM_SHARED((N_PAD,), jnp.float32),
        pltpu.VMEM_SHARED((N_PAD,), jnp.float32),
        pltpu.VMEM_SHARED((N_PAD,), jnp.float32),
        pltpu.VMEM_SHARED((2 * N_GRAPHS_C,), jnp.float32),
        pltpu.VMEM_SHARED((2 * N_GRAPHS_C,), jnp.float32),
    ],
)


@jax.jit
def kernel(x, edge_index, edge_weight, batch, W_rel, b_rel, W_root, W_fc, b_fc):
    ei_flat = edge_index.astype(jnp.int32).reshape(2 * N_EDGES_C)

    prel2, q2, bp2, par2 = pl.pallas_call(
        _proj_body,
        out_shape=[jax.ShapeDtypeStruct((1, N_PAD), jnp.float32),
                   jax.ShapeDtypeStruct((1, N_PAD), jnp.float32),
                   jax.ShapeDtypeStruct((1, N_PAD), jnp.int32),
                   jax.ShapeDtypeStruct((1, 128), jnp.float32)],
    )(x, W_rel, W_root, jnp.reshape(b_rel, (1, 1)),
      batch.astype(jnp.int32), W_fc, b_fc)

    y = _sc_call(prel2.reshape(N_PAD), q2.reshape(N_PAD),
                 ei_flat, edge_weight, bp2.reshape(N_PAD), par2.reshape(128))
    return y[:, None]
